# baseline (speedup 1.0000x reference)
import functools

import jax
import jax.numpy as jnp
from jax import lax
from jax.experimental import pallas as pl
from jax.experimental.pallas import tpu as pltpu
from jax.experimental.pallas import tpu_sc as plsc

_RADIUS = 0.2
_MAX_SAMPLES = 64
_BI = 256
_NC = 2
_NS = 16
_CR = 16


def _rank_tc_kernel(pcs_ref, g_ref, cnt_ref):
    i = pl.program_id(1)
    xall = pcs_ref[0]
    n = xall.shape[1]
    xblk = pcs_ref[0, :, pl.ds(i * _BI, _BI)]

    sq_all = xall[0] * xall[0] + xall[1] * xall[1] + xall[2] * xall[2]
    sq_blk = xblk[0] * xblk[0] + xblk[1] * xblk[1] + xblk[2] * xblk[2]
    dot = jnp.dot(xblk.T, xall, preferred_element_type=jnp.float32)
    d2 = (sq_blk[:, None] + sq_all[None, :]) - 2.0 * dot
    mask = d2 < _RADIUS * _RADIUS

    c = mask.astype(jnp.int16)
    k = 1
    while k < n:
        c = c + jnp.concatenate(
            [jnp.zeros((_BI, k), jnp.int16), c[:, : n - k]], axis=1)
        k *= 2

    g = jnp.where(mask & (c <= _MAX_SAMPLES), c, jnp.int16(0))
    h = n // 2
    lo = g[:, :h].astype(jnp.int32)
    hi = g[:, h:].astype(jnp.int32)
    g_ref[0] = lo | (hi << 16)
    cnt_ref[0] = c[:, n - 1:n].astype(jnp.int32)


def _sc_scatter_kernel(g_hbm, cnt_hbm, out_hbm,
                       buf0, buf1, cnt0, cnt1, ob0, ob1,
                       sg0, sg1, sc0, sc1, so0, so1):
    nb = g_hbm.shape[0]
    n_workers = _NC * _NS
    rows_per_worker = (nb * g_hbm.shape[1]) // n_workers
    workers_per_batch = n_workers // nb
    n_chunks = rows_per_worker // _CR
    wid = lax.axis_index("s") * _NC + lax.axis_index("c")
    batch = wid // workers_per_batch
    lr0 = (wid % workers_per_batch) * rows_per_worker

    bufs, cnts, obs = (buf0, buf1), (cnt0, cnt1), (ob0, ob1)
    sgs, scs, sos = (sg0, sg1), (sc0, sc1), (so0, so1)

    iota = lax.broadcasted_iota(jnp.int32, (16,), 0)
    zeros16 = jnp.zeros((16,), jnp.int32)

    def start_in(ci, u):
        r0 = lr0 + ci * _CR
        pltpu.make_async_copy(
            g_hbm.at[batch, pl.ds(r0, _CR)], bufs[u], sgs[u]).start()
        pltpu.make_async_copy(
            cnt_hbm.at[batch, pl.ds(r0, _CR)], cnts[u], scs[u]).start()

    def wait_in(u):
        pltpu.make_async_copy(
            g_hbm.at[batch, pl.ds(lr0, _CR)], bufs[u], sgs[u]).wait()
        pltpu.make_async_copy(
            cnt_hbm.at[batch, pl.ds(lr0, _CR)], cnts[u], scs[u]).wait()

    def process(u):
        buf, cntbuf, outbuf = bufs[u], cnts[u], obs[u]

        def row_body(r, _):
            rsplat = jnp.full((16,), r, jnp.int32)

            @plsc.parallel_loop(0, 1024 // 16, unroll=8)
            def _vec_body(k):
                v = buf[r, pl.ds(k * 16, 16)]
                a = v & 0xFFFF
                b = lax.shift_right_logical(v, 16)
                jv = iota + k * 16
                plsc.store_scatter(outbuf, [rsplat, a - 1], jv, mask=a > 0)
                plsc.store_scatter(outbuf, [rsplat, b - 1], jv + 1024, mask=b > 0)

            cntv = plsc.load_gather(cntbuf, [rsplat, zeros16])
            firstv = plsc.load_gather(outbuf, [rsplat, zeros16])
            for t in range(_MAX_SAMPLES // 16):
                sv = iota + (t * 16)
                cur = outbuf[r, pl.ds(t * 16, 16)]
                outbuf[r, pl.ds(t * 16, 16)] = jnp.where(sv < cntv, cur, firstv)
            return 0

        lax.fori_loop(0, _CR, row_body, 0)

    start_in(0, 0)
    start_in(1, 1)

    def pair_body(p, _):
        for u in range(2):
            ci = 2 * p + u
            wait_in(u)

            @pl.when(ci >= 2)
            def _():
                pltpu.make_async_copy(
                    obs[u], out_hbm.at[batch, pl.ds(lr0, _CR)], sos[u]).wait()

            process(u)

            @pl.when(ci + 2 < n_chunks)
            def _():
                start_in(ci + 2, u)

            pltpu.make_async_copy(
                obs[u], out_hbm.at[batch, pl.ds(lr0 + ci * _CR, _CR)],
                sos[u]).start()
        return 0

    lax.fori_loop(0, n_chunks // 2, pair_body, 0)
    for u in range(2):
        pltpu.make_async_copy(
            obs[u], out_hbm.at[batch, pl.ds(lr0, _CR)], sos[u]).wait()


@jax.jit
def kernel(pcs):
    b, _, n = pcs.shape
    tc = pl.pallas_call(
        _rank_tc_kernel,
        grid=(b, n // _BI),
        in_specs=[pl.BlockSpec((1, 3, n), lambda bb, ii: (bb, 0, 0))],
        out_specs=[
            pl.BlockSpec((1, _BI, n // 2), lambda bb, ii: (bb, ii, 0)),
            pl.BlockSpec((1, _BI, 1), lambda bb, ii: (bb, ii, 0)),
        ],
        out_shape=[
            jax.ShapeDtypeStruct((b, n, n // 2), jnp.int32),
            jax.ShapeDtypeStruct((b, n, 1), jnp.int32),
        ],
    )

    mesh = plsc.VectorSubcoreMesh(
        core_axis_name="c", subcore_axis_name="s",
        num_cores=_NC, num_subcores=_NS)
    sc = pl.kernel(
        _sc_scatter_kernel,
        out_type=jax.ShapeDtypeStruct((b, n, _MAX_SAMPLES), jnp.int32),
        mesh=mesh,
        scratch_types=[
            pltpu.VMEM((_CR, n // 2), jnp.int32),
            pltpu.VMEM((_CR, n // 2), jnp.int32),
            pltpu.VMEM((_CR, 1), jnp.int32),
            pltpu.VMEM((_CR, 1), jnp.int32),
            pltpu.VMEM((_CR, _MAX_SAMPLES), jnp.int32),
            pltpu.VMEM((_CR, _MAX_SAMPLES), jnp.int32),
            pltpu.SemaphoreType.DMA,
            pltpu.SemaphoreType.DMA,
            pltpu.SemaphoreType.DMA,
            pltpu.SemaphoreType.DMA,
            pltpu.SemaphoreType.DMA,
            pltpu.SemaphoreType.DMA,
        ],
        compiler_params=pltpu.CompilerParams(needs_layout_passes=False),
    )

    g, cnt = tc(pcs)
    out = sc(g, cnt)
    return out.astype(jnp.int64)


# speedup vs baseline: 1.3639x; 1.3639x over previous
import functools

import jax
import jax.numpy as jnp
from jax import lax
from jax.experimental import pallas as pl
from jax.experimental.pallas import tpu as pltpu
from jax.experimental.pallas import tpu_sc as plsc

_RADIUS = 0.2
_MAX_SAMPLES = 64
_BI = 256
_NC = 2
_NS = 16
_CR = 16


def _rank_tc_kernel(pcs_ref, g_ref, cnt_ref):
    i = pl.program_id(1)
    xall = pcs_ref[0]
    n = xall.shape[1]
    xblk = pcs_ref[0, :, pl.ds(i * _BI, _BI)]

    sq_all = xall[0] * xall[0] + xall[1] * xall[1] + xall[2] * xall[2]
    sq_blk = xblk[0] * xblk[0] + xblk[1] * xblk[1] + xblk[2] * xblk[2]
    dot = jnp.dot(xblk.T, xall, preferred_element_type=jnp.float32)
    d2 = (sq_blk[:, None] + sq_all[None, :]) - 2.0 * dot
    mask = d2 < _RADIUS * _RADIUS

    ch = 128
    nch = n // ch
    m_bf = mask.astype(jnp.bfloat16)
    r_io = jax.lax.broadcasted_iota(jnp.int32, (ch, ch), 0)
    c_io = jax.lax.broadcasted_iota(jnp.int32, (ch, ch), 1)
    tri = (r_io <= c_io).astype(jnp.bfloat16)
    locs = [jnp.dot(m_bf[:, t * ch:(t + 1) * ch], tri,
                    preferred_element_type=jnp.float32) for t in range(nch)]
    offs = [jnp.zeros((_BI, 1), jnp.float32)]
    for t in range(nch):
        offs.append(offs[t] + locs[t][:, ch - 1:ch])

    def rank_chunk(t):
        c_t = locs[t] + offs[t]
        ok = mask[:, t * ch:(t + 1) * ch] & (c_t <= _MAX_SAMPLES)
        return jnp.where(ok, c_t, 0.0).astype(jnp.int32)

    for t in range(nch // 2):
        packed = rank_chunk(t) | (rank_chunk(t + nch // 2) << 16)
        g_ref[0, :, t * ch:(t + 1) * ch] = packed
    cnt_ref[0] = offs[nch].astype(jnp.int32)


def _sc_scatter_kernel(g_hbm, cnt_hbm, out_hbm,
                       buf0, buf1, cnt0, cnt1, ob0, ob1,
                       sg0, sg1, sc0, sc1, so0, so1):
    nb = g_hbm.shape[0]
    n_workers = _NC * _NS
    rows_per_worker = (nb * g_hbm.shape[1]) // n_workers
    workers_per_batch = n_workers // nb
    n_chunks = rows_per_worker // _CR
    wid = lax.axis_index("s") * _NC + lax.axis_index("c")
    batch = wid // workers_per_batch
    lr0 = (wid % workers_per_batch) * rows_per_worker

    bufs, cnts, obs = (buf0, buf1), (cnt0, cnt1), (ob0, ob1)
    sgs, scs, sos = (sg0, sg1), (sc0, sc1), (so0, so1)

    iota = lax.broadcasted_iota(jnp.int32, (16,), 0)
    zeros16 = jnp.zeros((16,), jnp.int32)

    def start_in(ci, u):
        r0 = lr0 + ci * _CR
        pltpu.make_async_copy(
            g_hbm.at[batch, pl.ds(r0, _CR)], bufs[u], sgs[u]).start()
        pltpu.make_async_copy(
            cnt_hbm.at[batch, pl.ds(r0, _CR)], cnts[u], scs[u]).start()

    def wait_in(u):
        pltpu.make_async_copy(
            g_hbm.at[batch, pl.ds(lr0, _CR)], bufs[u], sgs[u]).wait()
        pltpu.make_async_copy(
            cnt_hbm.at[batch, pl.ds(lr0, _CR)], cnts[u], scs[u]).wait()

    def process(u):
        buf, cntbuf, outbuf = bufs[u], cnts[u], obs[u]

        def row_body(r, _):
            rsplat = jnp.full((16,), r, jnp.int32)

            @plsc.parallel_loop(0, 1024 // 16, unroll=8)
            def _vec_body(k):
                v = buf[r, pl.ds(k * 16, 16)]
                a = v & 0xFFFF
                b = lax.shift_right_logical(v, 16)
                jv = iota + k * 16
                plsc.store_scatter(outbuf, [rsplat, a - 1], jv, mask=a > 0)
                plsc.store_scatter(outbuf, [rsplat, b - 1], jv + 1024, mask=b > 0)

            cntv = plsc.load_gather(cntbuf, [rsplat, zeros16])
            firstv = plsc.load_gather(outbuf, [rsplat, zeros16])
            for t in range(_MAX_SAMPLES // 16):
                sv = iota + (t * 16)
                cur = outbuf[r, pl.ds(t * 16, 16)]
                outbuf[r, pl.ds(t * 16, 16)] = jnp.where(sv < cntv, cur, firstv)
            return 0

        lax.fori_loop(0, _CR, row_body, 0)

    start_in(0, 0)
    start_in(1, 1)

    def pair_body(p, _):
        for u in range(2):
            ci = 2 * p + u
            wait_in(u)

            @pl.when(ci >= 2)
            def _():
                pltpu.make_async_copy(
                    obs[u], out_hbm.at[batch, pl.ds(lr0, _CR)], sos[u]).wait()

            process(u)

            @pl.when(ci + 2 < n_chunks)
            def _():
                start_in(ci + 2, u)

            pltpu.make_async_copy(
                obs[u], out_hbm.at[batch, pl.ds(lr0 + ci * _CR, _CR)],
                sos[u]).start()
        return 0

    lax.fori_loop(0, n_chunks // 2, pair_body, 0)
    for u in range(2):
        pltpu.make_async_copy(
            obs[u], out_hbm.at[batch, pl.ds(lr0, _CR)], sos[u]).wait()


@jax.jit
def kernel(pcs):
    b, _, n = pcs.shape
    tc = pl.pallas_call(
        _rank_tc_kernel,
        grid=(b, n // _BI),
        in_specs=[pl.BlockSpec((1, 3, n), lambda bb, ii: (bb, 0, 0))],
        out_specs=[
            pl.BlockSpec((1, _BI, n // 2), lambda bb, ii: (bb, ii, 0)),
            pl.BlockSpec((1, _BI, 1), lambda bb, ii: (bb, ii, 0)),
        ],
        out_shape=[
            jax.ShapeDtypeStruct((b, n, n // 2), jnp.int32),
            jax.ShapeDtypeStruct((b, n, 1), jnp.int32),
        ],
    )

    mesh = plsc.VectorSubcoreMesh(
        core_axis_name="c", subcore_axis_name="s",
        num_cores=_NC, num_subcores=_NS)
    sc = pl.kernel(
        _sc_scatter_kernel,
        out_type=jax.ShapeDtypeStruct((b, n, _MAX_SAMPLES), jnp.int32),
        mesh=mesh,
        scratch_types=[
            pltpu.VMEM((_CR, n // 2), jnp.int32),
            pltpu.VMEM((_CR, n // 2), jnp.int32),
            pltpu.VMEM((_CR, 1), jnp.int32),
            pltpu.VMEM((_CR, 1), jnp.int32),
            pltpu.VMEM((_CR, _MAX_SAMPLES), jnp.int32),
            pltpu.VMEM((_CR, _MAX_SAMPLES), jnp.int32),
            pltpu.SemaphoreType.DMA,
            pltpu.SemaphoreType.DMA,
            pltpu.SemaphoreType.DMA,
            pltpu.SemaphoreType.DMA,
            pltpu.SemaphoreType.DMA,
            pltpu.SemaphoreType.DMA,
        ],
        compiler_params=pltpu.CompilerParams(needs_layout_passes=False),
    )

    g, cnt = tc(pcs)
    out = sc(g, cnt)
    return out.astype(jnp.int64)


# speedup vs baseline: 1.3820x; 1.0133x over previous
import functools

import jax
import jax.numpy as jnp
from jax import lax
from jax.experimental import pallas as pl
from jax.experimental.pallas import tpu as pltpu
from jax.experimental.pallas import tpu_sc as plsc

_RADIUS = 0.2
_MAX_SAMPLES = 64
_BI = 256
_NC = 2
_NS = 16
_CR = 32


def _rank_tc_kernel(pcs_ref, g_ref, cnt_ref):
    i = pl.program_id(1)
    xall = pcs_ref[0]
    n = xall.shape[1]
    xblk = pcs_ref[0, :, pl.ds(i * _BI, _BI)]

    sq_all = xall[0] * xall[0] + xall[1] * xall[1] + xall[2] * xall[2]
    sq_blk = xblk[0] * xblk[0] + xblk[1] * xblk[1] + xblk[2] * xblk[2]
    dot = jnp.dot(xblk.T, xall, preferred_element_type=jnp.float32)
    d2 = (sq_blk[:, None] + sq_all[None, :]) - 2.0 * dot
    mask = d2 < _RADIUS * _RADIUS

    ch = 128
    nch = n // ch
    m_bf = mask.astype(jnp.bfloat16)
    r_io = jax.lax.broadcasted_iota(jnp.int32, (ch, ch), 0)
    c_io = jax.lax.broadcasted_iota(jnp.int32, (ch, ch), 1)
    tri = (r_io <= c_io).astype(jnp.bfloat16)
    locs = [jnp.dot(m_bf[:, t * ch:(t + 1) * ch], tri,
                    preferred_element_type=jnp.float32) for t in range(nch)]
    offs = [jnp.zeros((_BI, 1), jnp.float32)]
    for t in range(nch):
        offs.append(offs[t] + locs[t][:, ch - 1:ch])

    def rank_chunk(t):
        c_t = locs[t] + offs[t]
        ok = mask[:, t * ch:(t + 1) * ch] & (c_t <= _MAX_SAMPLES)
        return jnp.where(ok, c_t - 1.0, 32768.0).astype(jnp.int32)

    for t in range(nch // 2):
        packed = rank_chunk(t) | (rank_chunk(t + nch // 2) << 16)
        g_ref[0, :, t * ch:(t + 1) * ch] = packed
    cnt_ref[0] = offs[nch].astype(jnp.int32)


def _sc_scatter_kernel(g_hbm, cnt_hbm, out_hbm,
                       buf0, buf1, cnt0, cnt1, ob0, ob1,
                       sg0, sg1, sc0, sc1, so0, so1):
    nb = g_hbm.shape[0]
    n_workers = _NC * _NS
    rows_per_worker = (nb * g_hbm.shape[1]) // n_workers
    workers_per_batch = n_workers // nb
    n_chunks = rows_per_worker // _CR
    wid = lax.axis_index("s") * _NC + lax.axis_index("c")
    batch = wid // workers_per_batch
    lr0 = (wid % workers_per_batch) * rows_per_worker

    bufs, cnts, obs = (buf0, buf1), (cnt0, cnt1), (ob0, ob1)
    sgs, scs, sos = (sg0, sg1), (sc0, sc1), (so0, so1)

    iota = lax.broadcasted_iota(jnp.int32, (16,), 0)
    zeros16 = jnp.zeros((16,), jnp.int32)

    def start_in(ci, u):
        r0 = lr0 + ci * _CR
        pltpu.make_async_copy(
            g_hbm.at[batch, pl.ds(r0, _CR)], bufs[u], sgs[u]).start()
        pltpu.make_async_copy(
            cnt_hbm.at[batch, pl.ds(r0, _CR)], cnts[u], scs[u]).start()

    def wait_in(u):
        pltpu.make_async_copy(
            g_hbm.at[batch, pl.ds(lr0, _CR)], bufs[u], sgs[u]).wait()
        pltpu.make_async_copy(
            cnt_hbm.at[batch, pl.ds(lr0, _CR)], cnts[u], scs[u]).wait()

    def process(u):
        buf, cntbuf, outbuf = bufs[u], cnts[u], obs[u]

        def row_body(r, _):
            rsplat = jnp.full((16,), r, jnp.int32)

            @plsc.parallel_loop(0, 1024 // 16, unroll=8)
            def _vec_body(k):
                v = buf[r, pl.ds(k * 16, 16)]
                a = v & 0xFFFF
                b = lax.shift_right_logical(v, 16)
                jv = iota + k * 16
                plsc.store_scatter(outbuf, [rsplat, a], jv, mask=a < 0x8000)
                plsc.store_scatter(outbuf, [rsplat, b], jv + 1024, mask=b < 0x8000)

            cntv = plsc.load_gather(cntbuf, [rsplat, zeros16])
            firstv = plsc.load_gather(outbuf, [rsplat, zeros16])
            for t in range(_MAX_SAMPLES // 16):
                sv = iota + (t * 16)
                cur = outbuf[r, pl.ds(t * 16, 16)]
                outbuf[r, pl.ds(t * 16, 16)] = jnp.where(sv < cntv, cur, firstv)
            return 0

        lax.fori_loop(0, _CR, row_body, 0)

    start_in(0, 0)
    start_in(1, 1)

    def pair_body(p, _):
        for u in range(2):
            ci = 2 * p + u
            wait_in(u)

            @pl.when(ci >= 2)
            def _():
                pltpu.make_async_copy(
                    obs[u], out_hbm.at[batch, pl.ds(lr0, _CR)], sos[u]).wait()

            process(u)

            @pl.when(ci + 2 < n_chunks)
            def _():
                start_in(ci + 2, u)

            pltpu.make_async_copy(
                obs[u], out_hbm.at[batch, pl.ds(lr0 + ci * _CR, _CR)],
                sos[u]).start()
        return 0

    lax.fori_loop(0, n_chunks // 2, pair_body, 0)
    for u in range(2):
        pltpu.make_async_copy(
            obs[u], out_hbm.at[batch, pl.ds(lr0, _CR)], sos[u]).wait()


@jax.jit
def kernel(pcs):
    b, _, n = pcs.shape
    tc = pl.pallas_call(
        _rank_tc_kernel,
        grid=(b, n // _BI),
        in_specs=[pl.BlockSpec((1, 3, n), lambda bb, ii: (bb, 0, 0))],
        out_specs=[
            pl.BlockSpec((1, _BI, n // 2), lambda bb, ii: (bb, ii, 0)),
            pl.BlockSpec((1, _BI, 1), lambda bb, ii: (bb, ii, 0)),
        ],
        out_shape=[
            jax.ShapeDtypeStruct((b, n, n // 2), jnp.int32),
            jax.ShapeDtypeStruct((b, n, 1), jnp.int32),
        ],
    )

    mesh = plsc.VectorSubcoreMesh(
        core_axis_name="c", subcore_axis_name="s",
        num_cores=_NC, num_subcores=_NS)
    sc = pl.kernel(
        _sc_scatter_kernel,
        out_type=jax.ShapeDtypeStruct((b, n, _MAX_SAMPLES), jnp.int32),
        mesh=mesh,
        scratch_types=[
            pltpu.VMEM((_CR, n // 2), jnp.int32),
            pltpu.VMEM((_CR, n // 2), jnp.int32),
            pltpu.VMEM((_CR, 1), jnp.int32),
            pltpu.VMEM((_CR, 1), jnp.int32),
            pltpu.VMEM((_CR, _MAX_SAMPLES), jnp.int32),
            pltpu.VMEM((_CR, _MAX_SAMPLES), jnp.int32),
            pltpu.SemaphoreType.DMA,
            pltpu.SemaphoreType.DMA,
            pltpu.SemaphoreType.DMA,
            pltpu.SemaphoreType.DMA,
            pltpu.SemaphoreType.DMA,
            pltpu.SemaphoreType.DMA,
        ],
        compiler_params=pltpu.CompilerParams(needs_layout_passes=False),
    )

    g, cnt = tc(pcs)
    out = sc(g, cnt)
    return out.astype(jnp.int64)


# speedup vs baseline: 1.6420x; 1.1882x over previous
import functools

import jax
import jax.numpy as jnp
from jax import lax
from jax.experimental import pallas as pl
from jax.experimental.pallas import tpu as pltpu
from jax.experimental.pallas import tpu_sc as plsc

_RADIUS = 0.2
_MAX_SAMPLES = 64
_BI = 256
_NC = 2
_NS = 16
_CR = 32


def _rank_tc_kernel(pcs_ref, g_ref, cnt_ref):
    i = pl.program_id(1)
    xall = pcs_ref[0]
    n = xall.shape[1]
    xblk = pcs_ref[0, :, pl.ds(i * _BI, _BI)]

    sq_all = xall[0] * xall[0] + xall[1] * xall[1] + xall[2] * xall[2]
    sq_blk = xblk[0] * xblk[0] + xblk[1] * xblk[1] + xblk[2] * xblk[2]
    dot = jnp.dot(xblk.T, xall, preferred_element_type=jnp.float32)
    d2 = (sq_blk[:, None] + sq_all[None, :]) - 2.0 * dot
    mask = d2 < _RADIUS * _RADIUS

    ch = 128
    nch = n // ch
    m_bf = mask.astype(jnp.bfloat16)
    r_io = jax.lax.broadcasted_iota(jnp.int32, (ch, ch), 0)
    c_io = jax.lax.broadcasted_iota(jnp.int32, (ch, ch), 1)
    tri = (r_io <= c_io).astype(jnp.bfloat16)
    locs = [jnp.dot(m_bf[:, t * ch:(t + 1) * ch], tri,
                    preferred_element_type=jnp.float32) for t in range(nch)]
    offs = [jnp.zeros((_BI, 1), jnp.float32)]
    for t in range(nch):
        offs.append(offs[t] + locs[t][:, ch - 1:ch])

    def rank_chunk(t):
        c_t = locs[t] + offs[t]
        ok = mask[:, t * ch:(t + 1) * ch] & (c_t <= _MAX_SAMPLES)
        return jnp.where(ok, c_t - 1.0, 32768.0).astype(jnp.int32)

    for t in range(nch // 2):
        packed = rank_chunk(t) | (rank_chunk(t + nch // 2) << 16)
        g_ref[0, :, t * ch:(t + 1) * ch] = packed
    cnt_ref[0] = offs[nch].astype(jnp.int32)


def _sc_scatter_kernel(g_hbm, cnt_hbm, out_hbm,
                       buf0, buf1, cnt0, cnt1, ob0, ob1,
                       sg0, sg1, sc0, sc1, so0, so1):
    nb = g_hbm.shape[0]
    n_workers = _NC * _NS
    rows_per_worker = (nb * g_hbm.shape[1]) // n_workers
    workers_per_batch = n_workers // nb
    n_chunks = rows_per_worker // _CR
    wid = lax.axis_index("s") * _NC + lax.axis_index("c")
    batch = wid // workers_per_batch
    lr0 = (wid % workers_per_batch) * rows_per_worker

    bufs, cnts, obs = (buf0, buf1), (cnt0, cnt1), (ob0, ob1)
    sgs, scs, sos = (sg0, sg1), (sc0, sc1), (so0, so1)

    iota = lax.broadcasted_iota(jnp.int32, (16,), 0)
    zeros16 = jnp.zeros((16,), jnp.int32)

    def start_in(ci, u):
        r0 = lr0 + ci * _CR
        pltpu.make_async_copy(
            g_hbm.at[batch, pl.ds(r0, _CR)], bufs[u], sgs[u]).start()
        pltpu.make_async_copy(
            cnt_hbm.at[batch, pl.ds(r0, _CR)], cnts[u], scs[u]).start()

    def wait_in(u):
        pltpu.make_async_copy(
            g_hbm.at[batch, pl.ds(lr0, _CR)], bufs[u], sgs[u]).wait()
        pltpu.make_async_copy(
            cnt_hbm.at[batch, pl.ds(lr0, _CR)], cnts[u], scs[u]).wait()

    def process(u):
        buf, cntbuf, outbuf = bufs[u], cnts[u], obs[u]

        def row_body(r, _):
            rsplat = jnp.full((16,), r, jnp.int32)

            @plsc.parallel_loop(0, 1024 // 16, unroll=8)
            def _vec_body(k):
                v = buf[r, pl.ds(k * 16, 16)]
                a = v & 0xFFFF
                b = lax.shift_right_logical(v, 16)
                jv = iota + k * 16
                plsc.store_scatter(outbuf, [rsplat, a], jv, mask=a < 0x8000)
                plsc.store_scatter(outbuf, [rsplat, b], jv + 1024, mask=b < 0x8000)

            cntv = plsc.load_gather(cntbuf, [rsplat, zeros16])
            firstv = plsc.load_gather(outbuf, [rsplat, zeros16])
            for t in range(_MAX_SAMPLES // 16):
                sv = iota + (t * 16)
                cur = outbuf[r, pl.ds(t * 16, 16)]
                outbuf[r, pl.ds(t * 16, 16)] = jnp.where(sv < cntv, cur, firstv)
            return 0

        lax.fori_loop(0, _CR, row_body, 0)

    start_in(0, 0)
    start_in(1, 1)

    def pair_body(p, _):
        for u in range(2):
            ci = 2 * p + u
            wait_in(u)

            @pl.when(ci >= 2)
            def _():
                pltpu.make_async_copy(
                    obs[u], out_hbm.at[batch, pl.ds(lr0, _CR)], sos[u]).wait()

            process(u)

            @pl.when(ci + 2 < n_chunks)
            def _():
                start_in(ci + 2, u)

            pltpu.make_async_copy(
                obs[u], out_hbm.at[batch, pl.ds(lr0 + ci * _CR, _CR)],
                sos[u]).start()
        return 0

    lax.fori_loop(0, n_chunks // 2, pair_body, 0)
    for u in range(2):
        pltpu.make_async_copy(
            obs[u], out_hbm.at[batch, pl.ds(lr0, _CR)], sos[u]).wait()


@jax.jit
def kernel(pcs):
    b, _, n = pcs.shape
    bh = b // 2
    tc = pl.pallas_call(
        _rank_tc_kernel,
        grid=(bh, n // _BI),
        in_specs=[pl.BlockSpec((1, 3, n), lambda bb, ii: (bb, 0, 0))],
        out_specs=[
            pl.BlockSpec((1, _BI, n // 2), lambda bb, ii: (bb, ii, 0)),
            pl.BlockSpec((1, _BI, 1), lambda bb, ii: (bb, ii, 0)),
        ],
        out_shape=[
            jax.ShapeDtypeStruct((bh, n, n // 2), jnp.int32),
            jax.ShapeDtypeStruct((bh, n, 1), jnp.int32),
        ],
    )

    mesh = plsc.VectorSubcoreMesh(
        core_axis_name="c", subcore_axis_name="s",
        num_cores=_NC, num_subcores=_NS)
    sc = pl.kernel(
        _sc_scatter_kernel,
        out_type=jax.ShapeDtypeStruct((bh, n, _MAX_SAMPLES), jnp.int32),
        mesh=mesh,
        scratch_types=[
            pltpu.VMEM((_CR, n // 2), jnp.int32),
            pltpu.VMEM((_CR, n // 2), jnp.int32),
            pltpu.VMEM((_CR, 1), jnp.int32),
            pltpu.VMEM((_CR, 1), jnp.int32),
            pltpu.VMEM((_CR, _MAX_SAMPLES), jnp.int32),
            pltpu.VMEM((_CR, _MAX_SAMPLES), jnp.int32),
            pltpu.SemaphoreType.DMA,
            pltpu.SemaphoreType.DMA,
            pltpu.SemaphoreType.DMA,
            pltpu.SemaphoreType.DMA,
            pltpu.SemaphoreType.DMA,
            pltpu.SemaphoreType.DMA,
        ],
        compiler_params=pltpu.CompilerParams(needs_layout_passes=False),
    )

    outs = []
    for hh in range(2):
        g, cnt = tc(pcs[hh * (b // 2):(hh + 1) * (b // 2)])
        outs.append(sc(g, cnt))
    return jnp.concatenate(outs, axis=0).astype(jnp.int64)


# speedup vs baseline: 1.7229x; 1.0493x over previous
import functools

import jax
import jax.numpy as jnp
from jax import lax
from jax.experimental import pallas as pl
from jax.experimental.pallas import tpu as pltpu
from jax.experimental.pallas import tpu_sc as plsc

_RADIUS = 0.2
_MAX_SAMPLES = 64
_BI = 256
_NC = 2
_NS = 16
_CR = 32


def _rank_tc_kernel(pcs_ref, g_ref, cnt_ref):
    i = pl.program_id(1)
    xall = pcs_ref[0]
    n = xall.shape[1]
    xblk = pcs_ref[0, :, pl.ds(i * _BI, _BI)]

    sq_all = xall[0] * xall[0] + xall[1] * xall[1] + xall[2] * xall[2]
    sq_blk = xblk[0] * xblk[0] + xblk[1] * xblk[1] + xblk[2] * xblk[2]
    dot = jnp.dot(xblk.T, xall, preferred_element_type=jnp.float32)
    d2 = (sq_blk[:, None] + sq_all[None, :]) - 2.0 * dot
    mask = d2 < _RADIUS * _RADIUS

    ch = 128
    nch = n // ch
    m_bf = mask.astype(jnp.bfloat16)
    r_io = jax.lax.broadcasted_iota(jnp.int32, (ch, ch), 0)
    c_io = jax.lax.broadcasted_iota(jnp.int32, (ch, ch), 1)
    tri = (r_io <= c_io).astype(jnp.bfloat16)
    locs = [jnp.dot(m_bf[:, t * ch:(t + 1) * ch], tri,
                    preferred_element_type=jnp.float32) for t in range(nch)]
    offs = [jnp.zeros((_BI, 1), jnp.float32)]
    for t in range(nch):
        offs.append(offs[t] + locs[t][:, ch - 1:ch])

    def rank_chunk(t):
        c_t = locs[t] + offs[t]
        ok = mask[:, t * ch:(t + 1) * ch] & (c_t <= _MAX_SAMPLES)
        return jnp.where(ok, c_t - 1.0, 32768.0).astype(jnp.int32)

    for t in range(nch // 2):
        packed = rank_chunk(t) | (rank_chunk(t + nch // 2) << 16)
        g_ref[0, :, t * ch:(t + 1) * ch] = packed
    cnt_ref[0] = offs[nch].astype(jnp.int32)


def _sc_scatter_kernel(g_hbm, cnt_hbm, out_hbm,
                       buf0, buf1, cnt0, cnt1, ob0, ob1,
                       sg0, sg1, sc0, sc1, so0, so1):
    nb = g_hbm.shape[0]
    n_workers = _NC * _NS
    rows_per_worker = (nb * g_hbm.shape[1]) // n_workers
    workers_per_batch = n_workers // nb
    n_chunks = rows_per_worker // _CR
    wid = lax.axis_index("s") * _NC + lax.axis_index("c")
    batch = wid // workers_per_batch
    lr0 = (wid % workers_per_batch) * rows_per_worker

    bufs, cnts, obs = (buf0, buf1), (cnt0, cnt1), (ob0, ob1)
    sgs, scs, sos = (sg0, sg1), (sc0, sc1), (so0, so1)

    iota = lax.broadcasted_iota(jnp.int32, (16,), 0)
    zeros16 = jnp.zeros((16,), jnp.int32)

    def start_in(ci, u):
        r0 = lr0 + ci * _CR
        pltpu.make_async_copy(
            g_hbm.at[batch, pl.ds(r0, _CR)], bufs[u], sgs[u]).start()
        pltpu.make_async_copy(
            cnt_hbm.at[batch, pl.ds(r0, _CR)], cnts[u], scs[u]).start()

    def wait_in(u):
        pltpu.make_async_copy(
            g_hbm.at[batch, pl.ds(lr0, _CR)], bufs[u], sgs[u]).wait()
        pltpu.make_async_copy(
            cnt_hbm.at[batch, pl.ds(lr0, _CR)], cnts[u], scs[u]).wait()

    def process(u):
        buf, cntbuf, outbuf = bufs[u], cnts[u], obs[u]

        def row_body(r, _):
            rsplat = jnp.full((16,), r, jnp.int32)

            @plsc.parallel_loop(0, 1024 // 16, unroll=8)
            def _vec_body(k):
                v = buf[r, pl.ds(k * 16, 16)]
                a = v & 0xFFFF
                b = lax.shift_right_logical(v, 16)
                jv = iota + k * 16
                plsc.store_scatter(outbuf, [rsplat, a], jv, mask=a < 0x8000)
                plsc.store_scatter(outbuf, [rsplat, b], jv + 1024, mask=b < 0x8000)

            cntv = plsc.load_gather(cntbuf, [rsplat, zeros16])
            firstv = plsc.load_gather(outbuf, [rsplat, zeros16])
            for t in range(_MAX_SAMPLES // 16):
                sv = iota + (t * 16)
                cur = outbuf[r, pl.ds(t * 16, 16)]
                outbuf[r, pl.ds(t * 16, 16)] = jnp.where(sv < cntv, cur, firstv)
            return 0

        lax.fori_loop(0, _CR, row_body, 0)

    start_in(0, 0)
    start_in(1, 1)

    def pair_body(p, _):
        for u in range(2):
            ci = 2 * p + u
            wait_in(u)

            @pl.when(ci >= 2)
            def _():
                pltpu.make_async_copy(
                    obs[u], out_hbm.at[batch, pl.ds(lr0, _CR)], sos[u]).wait()

            process(u)

            @pl.when(ci + 2 < n_chunks)
            def _():
                start_in(ci + 2, u)

            pltpu.make_async_copy(
                obs[u], out_hbm.at[batch, pl.ds(lr0 + ci * _CR, _CR)],
                sos[u]).start()
        return 0

    lax.fori_loop(0, n_chunks // 2, pair_body, 0)
    for u in range(2):
        pltpu.make_async_copy(
            obs[u], out_hbm.at[batch, pl.ds(lr0, _CR)], sos[u]).wait()


@jax.jit
def kernel(pcs):
    b, _, n = pcs.shape
    bh = b // 4
    tc = pl.pallas_call(
        _rank_tc_kernel,
        grid=(bh, n // _BI),
        in_specs=[pl.BlockSpec((1, 3, n), lambda bb, ii: (bb, 0, 0))],
        out_specs=[
            pl.BlockSpec((1, _BI, n // 2), lambda bb, ii: (bb, ii, 0)),
            pl.BlockSpec((1, _BI, 1), lambda bb, ii: (bb, ii, 0)),
        ],
        out_shape=[
            jax.ShapeDtypeStruct((bh, n, n // 2), jnp.int32),
            jax.ShapeDtypeStruct((bh, n, 1), jnp.int32),
        ],
    )

    mesh = plsc.VectorSubcoreMesh(
        core_axis_name="c", subcore_axis_name="s",
        num_cores=_NC, num_subcores=_NS)
    sc = pl.kernel(
        _sc_scatter_kernel,
        out_type=jax.ShapeDtypeStruct((bh, n, _MAX_SAMPLES), jnp.int32),
        mesh=mesh,
        scratch_types=[
            pltpu.VMEM((_CR, n // 2), jnp.int32),
            pltpu.VMEM((_CR, n // 2), jnp.int32),
            pltpu.VMEM((_CR, 1), jnp.int32),
            pltpu.VMEM((_CR, 1), jnp.int32),
            pltpu.VMEM((_CR, _MAX_SAMPLES), jnp.int32),
            pltpu.VMEM((_CR, _MAX_SAMPLES), jnp.int32),
            pltpu.SemaphoreType.DMA,
            pltpu.SemaphoreType.DMA,
            pltpu.SemaphoreType.DMA,
            pltpu.SemaphoreType.DMA,
            pltpu.SemaphoreType.DMA,
            pltpu.SemaphoreType.DMA,
        ],
        compiler_params=pltpu.CompilerParams(needs_layout_passes=False),
    )

    outs = []
    for hh in range(4):
        g, cnt = tc(pcs[hh * (b // 4):(hh + 1) * (b // 4)])
        outs.append(sc(g, cnt))
    return jnp.concatenate(outs, axis=0).astype(jnp.int64)


# speedup vs baseline: 1.7234x; 1.0003x over previous
import jax
import jax.numpy as jnp
from jax import lax
from jax.experimental import pallas as pl
from jax.experimental.pallas import tpu as pltpu
from jax.experimental.pallas import tpu_sc as plsc

_RADIUS = 0.2
_MAX_SAMPLES = 64
_BI = 256
_NC = 2
_NS = 16
_CR = 32


def _rank_tc_kernel(pcs_ref, g_ref, cnt_ref):
    i = pl.program_id(1)
    xall = pcs_ref[0]
    n = xall.shape[1]
    xblk = pcs_ref[0, :, pl.ds(i * _BI, _BI)]

    sq_all = xall[0] * xall[0] + xall[1] * xall[1] + xall[2] * xall[2]
    sq_blk = xblk[0] * xblk[0] + xblk[1] * xblk[1] + xblk[2] * xblk[2]
    dot = jnp.dot(xblk.T, xall, preferred_element_type=jnp.float32)
    d2 = (sq_blk[:, None] + sq_all[None, :]) - 2.0 * dot
    mask = d2 < _RADIUS * _RADIUS

    ch = 128
    nch = n // ch
    m_bf = mask.astype(jnp.bfloat16)
    r_io = jax.lax.broadcasted_iota(jnp.int32, (ch, ch), 0)
    c_io = jax.lax.broadcasted_iota(jnp.int32, (ch, ch), 1)
    tri = (r_io <= c_io).astype(jnp.bfloat16)
    locs = [jnp.dot(m_bf[:, t * ch:(t + 1) * ch], tri,
                    preferred_element_type=jnp.float32) for t in range(nch)]
    offs = [jnp.zeros((_BI, 1), jnp.float32)]
    for t in range(nch):
        offs.append(offs[t] + locs[t][:, ch - 1:ch])

    def rank_chunk(t):
        c_t = locs[t] + offs[t]
        ok = mask[:, t * ch:(t + 1) * ch] & (c_t <= _MAX_SAMPLES)
        return jnp.where(ok, c_t - 1.0, 32768.0).astype(jnp.int32)

    for t in range(nch // 2):
        packed = rank_chunk(t) | (rank_chunk(t + nch // 2) << 16)
        g_ref[0, :, t * ch:(t + 1) * ch] = packed
    cnt_ref[0] = offs[nch].astype(jnp.int32)


def _sc_scatter_kernel(g_hbm, cnt_hbm, out_hbm,
                       buf0, buf1, cnt0, cnt1, ob0, ob1,
                       sg0, sg1, sc0, sc1, so0, so1):
    nb = g_hbm.shape[0]
    n_workers = _NC * _NS
    rows_per_worker = (nb * g_hbm.shape[1]) // n_workers
    workers_per_batch = n_workers // nb
    n_chunks = rows_per_worker // _CR
    wid = lax.axis_index("s") * _NC + lax.axis_index("c")
    batch = wid // workers_per_batch
    lr0 = (wid % workers_per_batch) * rows_per_worker

    bufs, cnts, obs = (buf0, buf1), (cnt0, cnt1), (ob0, ob1)
    sgs, scs, sos = (sg0, sg1), (sc0, sc1), (so0, so1)

    iota = lax.broadcasted_iota(jnp.int32, (16,), 0)
    zeros16 = jnp.zeros((16,), jnp.int32)

    def start_in(ci, u):
        r0 = lr0 + ci * _CR
        pltpu.make_async_copy(
            g_hbm.at[batch, pl.ds(r0, _CR)], bufs[u], sgs[u]).start()
        pltpu.make_async_copy(
            cnt_hbm.at[batch, pl.ds(r0, _CR)], cnts[u], scs[u]).start()

    def wait_in(u):
        pltpu.make_async_copy(
            g_hbm.at[batch, pl.ds(lr0, _CR)], bufs[u], sgs[u]).wait()
        pltpu.make_async_copy(
            cnt_hbm.at[batch, pl.ds(lr0, _CR)], cnts[u], scs[u]).wait()

    def process(u):
        buf, cntbuf, outbuf = bufs[u], cnts[u], obs[u]

        def row_body(r, _):
            rsplat = jnp.full((16,), r, jnp.int32)

            @plsc.parallel_loop(0, 1024 // 16, unroll=8)
            def _vec_body(k):
                v = buf[r, pl.ds(k * 16, 16)]
                a = v & 0xFFFF
                b = lax.shift_right_logical(v, 16)
                jv = iota + k * 16
                plsc.store_scatter(outbuf, [rsplat, a], jv, mask=a < 0x8000)
                plsc.store_scatter(outbuf, [rsplat, b], jv + 1024, mask=b < 0x8000)

            cntv = plsc.load_gather(cntbuf, [rsplat, zeros16])
            firstv = plsc.load_gather(outbuf, [rsplat, zeros16])
            for t in range(_MAX_SAMPLES // 16):
                sv = iota + (t * 16)
                cur = outbuf[r, pl.ds(t * 16, 16)]
                outbuf[r, pl.ds(t * 16, 16)] = jnp.where(sv < cntv, cur, firstv)
            return 0

        lax.fori_loop(0, _CR, row_body, 0)

    start_in(0, 0)
    start_in(1, 1)

    def pair_body(p, _):
        for u in range(2):
            ci = 2 * p + u
            wait_in(u)

            @pl.when(ci >= 2)
            def _():
                pltpu.make_async_copy(
                    obs[u], out_hbm.at[batch, pl.ds(lr0, _CR)], sos[u]).wait()

            process(u)

            @pl.when(ci + 2 < n_chunks)
            def _():
                start_in(ci + 2, u)

            pltpu.make_async_copy(
                obs[u], out_hbm.at[batch, pl.ds(lr0 + ci * _CR, _CR)],
                sos[u]).start()
        return 0

    lax.fori_loop(0, n_chunks // 2, pair_body, 0)
    for u in range(2):
        pltpu.make_async_copy(
            obs[u], out_hbm.at[batch, pl.ds(lr0, _CR)], sos[u]).wait()


@jax.jit
def kernel(pcs):
    b, _, n = pcs.shape
    bh = b // 4
    tc = pl.pallas_call(
        _rank_tc_kernel,
        grid=(bh, n // _BI),
        in_specs=[pl.BlockSpec((1, 3, n), lambda bb, ii: (bb, 0, 0))],
        out_specs=[
            pl.BlockSpec((1, _BI, n // 2), lambda bb, ii: (bb, ii, 0)),
            pl.BlockSpec((1, _BI, 1), lambda bb, ii: (bb, ii, 0)),
        ],
        out_shape=[
            jax.ShapeDtypeStruct((bh, n, n // 2), jnp.int32),
            jax.ShapeDtypeStruct((bh, n, 1), jnp.int32),
        ],
    )

    mesh = plsc.VectorSubcoreMesh(
        core_axis_name="c", subcore_axis_name="s",
        num_cores=_NC, num_subcores=_NS)
    sc = pl.kernel(
        _sc_scatter_kernel,
        out_type=jax.ShapeDtypeStruct((bh, n, _MAX_SAMPLES), jnp.int32),
        mesh=mesh,
        scratch_types=[
            pltpu.VMEM((_CR, n // 2), jnp.int32),
            pltpu.VMEM((_CR, n // 2), jnp.int32),
            pltpu.VMEM((_CR, 1), jnp.int32),
            pltpu.VMEM((_CR, 1), jnp.int32),
            pltpu.VMEM((_CR, _MAX_SAMPLES), jnp.int32),
            pltpu.VMEM((_CR, _MAX_SAMPLES), jnp.int32),
            pltpu.SemaphoreType.DMA,
            pltpu.SemaphoreType.DMA,
            pltpu.SemaphoreType.DMA,
            pltpu.SemaphoreType.DMA,
            pltpu.SemaphoreType.DMA,
            pltpu.SemaphoreType.DMA,
        ],
        compiler_params=pltpu.CompilerParams(needs_layout_passes=False),
    )

    outs = []
    for hh in range(4):
        g, cnt = tc(pcs[hh * (b // 4):(hh + 1) * (b // 4)])
        outs.append(sc(g, cnt))
    return jnp.concatenate(outs, axis=0).astype(jnp.int64)
